# baseline (speedup 1.0000x reference)
import functools

import jax
import jax.numpy as jnp
from jax import lax
from jax.experimental import pallas as pl
from jax.experimental.pallas import tpu as pltpu
import jax.experimental.pallas.tpu_sc as plsc

N = 10000
H = 256
HC = 128
E = 160000
L = 40000
NC, NS = 2, 16
EB = 80
EPT = E // NS
AEB = 125
ANB = EPT // AEB
NBAT = EPT // EB
RCH = 80
NRCH = N // RCH
LPAD = 40960
LPT = LPAD // (NC * NS)
LBAT = LPT // EB

_MESH = plsc.VectorSubcoreMesh(core_axis_name="c", subcore_axis_name="s")
_UNTILED = pltpu.CompilerParams(use_tc_tiling_on_sc=False)


def _count_body(dst_a, dst_b, zeros_s, out, didx, ones_v, stage, hist):
    c = lax.axis_index("c")
    s = lax.axis_index("s")

    def fill_ones(i, _):
        ones_v[i, :] = jnp.ones((16,), jnp.float32)
        return 0

    lax.fori_loop(0, AEB, fill_ones, 0)

    @pl.when(c == 0)
    def _():
        pltpu.sync_copy(dst_a.at[s], didx)

    @pl.when(c == 1)
    def _():
        pltpu.sync_copy(dst_b.at[s], didx)

    nch = jnp.where(s < NRCH - 7 * NS, 8, 7)

    def zero(k, _):
        pltpu.sync_copy(zeros_s, hist.at[pl.ds((s + k * NS) * RCH, RCH)])
        return 0

    lax.fori_loop(0, nch, zero, 0)
    plsc.subcore_barrier()

    def scat(i, _):
        pltpu.sync_copy(ones_v, hist.at[didx.at[i]], add=True)
        return 0

    lax.fori_loop(0, ANB, scat, 0)
    plsc.subcore_barrier()

    def wb(k, _):
        off = (s + k * NS) * RCH
        pltpu.sync_copy(hist.at[pl.ds(off, RCH)], stage)
        pltpu.sync_copy(stage, out.at[c, pl.ds(off, RCH)])
        return 0

    lax.fori_loop(0, nch, wb, 0)


@functools.partial(
    pl.kernel,
    out_type=jax.ShapeDtypeStruct((NC, N, 16), jnp.float32),
    mesh=_MESH,
    compiler_params=_UNTILED,
    scratch_types=[
        pltpu.VMEM((ANB, AEB), jnp.int32),
        pltpu.VMEM((AEB, 16), jnp.float32),
        pltpu.VMEM((RCH, 16), jnp.float32),
        pltpu.VMEM_SHARED((N, 16), jnp.float32),
    ],
)
def _sc_counts(dst_a, dst_b, zeros_s, out, didx, ones_v, stage, hist):
    _count_body(dst_a, dst_b, zeros_s, out, didx, ones_v, stage, hist)


QC = 64


def _agg_body(xa0, xa1, xa2, xa3, xb0, xb1, xb2, xb3,
              srca, dsta, srcb, dstb, zeros_b,
              oa0, oa1, oa2, oa3, ob0, ob1, ob2, ob3,
              sidx, didx, *rest):
    bufs = rest[0:8]
    zb = rest[8]
    acc = rest[9]
    gsems = rest[10:18]
    ssems = rest[18:26]
    nb = len(bufs)
    c = lax.axis_index("c")
    s = lax.axis_index("s")

    @pl.when(c == 0)
    def _():
        pltpu.sync_copy(srca.at[s], sidx)
        pltpu.sync_copy(dsta.at[s], didx)

    @pl.when(c == 1)
    def _():
        pltpu.sync_copy(srcb.at[s], sidx)
        pltpu.sync_copy(dstb.at[s], didx)

    nch = jnp.where(s < NRCH - 7 * NS, 8, 7)

    def zero(k, _):
        pltpu.sync_copy(zeros_b, acc.at[pl.ds((s + k * NS) * RCH, RCH)])
        return 0

    for p, (xa, xb, oa, ob) in enumerate(((xa0, xb0, oa0, ob0),
                                          (xa1, xb1, oa1, ob1),
                                          (xa2, xb2, oa2, ob2),
                                          (xa3, xb3, oa3, ob3))):
        lax.fori_loop(0, nch, zero, 0)
        plsc.subcore_barrier()

        def start_gather(i, buf, gsem):
            @pl.when(c == 0)
            def _():
                pltpu.async_copy(xa.at[sidx.at[i]], buf, gsem)

            @pl.when(c == 1)
            def _():
                pltpu.async_copy(xb.at[sidx.at[i]], buf, gsem)

        def wait_gather(buf, gsem):
            pltpu.make_async_copy(xa.at[sidx.at[0]], buf, gsem).wait()

        def start_scatter(i, buf, ssem):
            pltpu.async_copy(buf, acc.at[didx.at[i]], ssem, add=True)

        def wait_scatter(i, buf, ssem):
            pltpu.make_async_copy(buf, acc.at[didx.at[i]], ssem).wait()

        for b in range(nb):
            start_gather(b, bufs[b], gsems[b])

        def bodyn(k, _):
            i0 = nb * k
            for b in range(nb):
                wait_gather(bufs[b], gsems[b])
                start_scatter(i0 + b, bufs[b], ssems[b])
            for b in range(nb):
                wait_scatter(i0 + b, bufs[b], ssems[b])

                @pl.when(i0 + b + nb < ANB)
                def _():
                    start_gather(i0 + b + nb, bufs[b], gsems[b])

            return 0

        lax.fori_loop(0, ANB // nb, bodyn, 0)
        plsc.subcore_barrier()

        def wb(k, _):
            off = (s + k * NS) * RCH
            pltpu.sync_copy(acc.at[pl.ds(off, RCH)], zb)

            @pl.when(c == 0)
            def _():
                pltpu.sync_copy(zb, oa.at[pl.ds(off, RCH)])

            @pl.when(c == 1)
            def _():
                pltpu.sync_copy(zb, ob.at[pl.ds(off, RCH)])

            return 0

        lax.fori_loop(0, nch, wb, 0)
        if p < 3:
            plsc.subcore_barrier()


_QSDS = jax.ShapeDtypeStruct((N, QC), jnp.float32)


@functools.partial(
    pl.kernel,
    out_type=(_QSDS,) * 8,
    mesh=_MESH,
    compiler_params=_UNTILED,
    scratch_types=[
        pltpu.VMEM((ANB, AEB), jnp.int32),
        pltpu.VMEM((ANB, AEB), jnp.int32),
    ] + [pltpu.VMEM((AEB, QC), jnp.float32)] * 8 + [
        pltpu.VMEM((RCH, QC), jnp.float32),
        pltpu.VMEM_SHARED((N, QC), jnp.float32),
    ] + [pltpu.SemaphoreType.DMA] * 16,
)
def _sc_agg(*args):
    _agg_body(*args)


def _dec_body(p_t, q_t, row3, col3, w2, b2v, out,
              ridx, cidx, w2v, b2s, pr0, qr0, pr1, qr1, outv,
              semp0, semq0, semp1, semq1):
    c = lax.axis_index("c")
    s = lax.axis_index("s")
    wid = s * NC + c

    pltpu.sync_copy(row3.at[wid], ridx)
    pltpu.sync_copy(col3.at[wid], cidx)
    pltpu.sync_copy(w2, w2v)
    pltpu.sync_copy(b2v, b2s)

    prs = (pr0, pr1)
    qrs = (qr0, qr1)
    psems = (semp0, semp1)
    qsems = (semq0, semq1)

    def start(j, b):
        pltpu.async_copy(p_t.at[ridx.at[pl.ds(j * EB, EB)]], prs[b], psems[b])
        pltpu.async_copy(q_t.at[cidx.at[pl.ds(j * EB, EB)]], qrs[b], qsems[b])

    def wait(b):
        pltpu.make_async_copy(p_t.at[ridx.at[pl.ds(0, EB)]],
                              prs[b], psems[b]).wait()
        pltpu.make_async_copy(q_t.at[cidx.at[pl.ds(0, EB)]],
                              qrs[b], qsems[b]).wait()

    lanes = lax.iota(jnp.int32, 16)

    def compute(j, pr, qr):
        def edge(e, vec):
            acc = b2s[...]
            for h in range(H // 16):
                pch = pr[e, pl.ds(h * 16, 16)]
                qch = qr[e, pl.ds(h * 16, 16)]
                g = jnp.maximum(pch + qch, 0.0)
                acc = acc + g * w2v[pl.ds(h * 16, 16)]
            lane = lax.rem(e, 16)
            vec = jnp.where(lanes == lane, jnp.sum(acc), vec)

            @pl.when(lane == 15)
            def _():
                outv[pl.ds(j * EB + e - 15, 16)] = vec

            return vec

        lax.fori_loop(0, EB, edge, jnp.zeros((16,), jnp.float32))

    start(0, 0)

    def pair(k, _):
        j0 = 2 * k
        start(j0 + 1, 1)
        wait(0)
        compute(j0, pr0, qr0)

        @pl.when(j0 + 2 < LBAT)
        def _():
            start(j0 + 2, 0)

        wait(1)
        compute(j0 + 1, pr1, qr1)
        return 0

    lax.fori_loop(0, LBAT // 2, pair, 0)
    pltpu.sync_copy(outv, out.at[pl.ds(wid * LPT, LPT)])


@functools.partial(
    pl.kernel,
    out_type=jax.ShapeDtypeStruct((LPAD,), jnp.float32),
    mesh=_MESH,
    compiler_params=pltpu.CompilerParams(needs_layout_passes=False),
    scratch_types=[
        pltpu.VMEM((LPT,), jnp.int32),
        pltpu.VMEM((LPT,), jnp.int32),
        pltpu.VMEM((H,), jnp.float32),
        pltpu.VMEM((16,), jnp.float32),
        pltpu.VMEM((EB, H), jnp.float32),
        pltpu.VMEM((EB, H), jnp.float32),
        pltpu.VMEM((EB, H), jnp.float32),
        pltpu.VMEM((EB, H), jnp.float32),
        pltpu.VMEM((LPT,), jnp.float32),
        pltpu.SemaphoreType.DMA,
        pltpu.SemaphoreType.DMA,
        pltpu.SemaphoreType.DMA,
        pltpu.SemaphoreType.DMA,
    ],
)
def _sc_decoder(p_t, q_t, row3, col3, w2, b2v, out,
                ridx, cidx, w2v, b2s, pr0, qr0, pr1, qr1, outv,
                semp0, semq0, semp1, semq1):
    _dec_body(p_t, q_t, row3, col3, w2, b2v, out,
              ridx, cidx, w2v, b2s, pr0, qr0, pr1, qr1, outv,
              semp0, semq0, semp1, semq1)


_RB = 1000


def _dot(a, b):
    return jnp.dot(a, b, preferred_element_type=jnp.float32)


_QBLK = pl.BlockSpec((_RB, QC), lambda i: (i, 0))
_WBLK = pl.BlockSpec((H, H), lambda i: (0, 0))
_BBLK = pl.BlockSpec((1, H), lambda i: (0, 0))
_QOUT = (_QBLK, _QBLK, _QBLK, _QBLK)
_QSHAPE = tuple(jax.ShapeDtypeStruct((N, QC), jnp.float32) for _ in range(4))


def _split4(out_refs, y):
    for k in range(4):
        out_refs[k][...] = y[:, k * QC:(k + 1) * QC]


def _proj_body(x_ref, w_ref, b_ref, *out_refs):
    _split4(out_refs, _dot(x_ref[...], w_ref[...]) + b_ref[...])


def _tc_proj(x, w, b):
    return pl.pallas_call(
        _proj_body,
        grid=(N // _RB,),
        in_specs=[
            pl.BlockSpec((_RB, H), lambda i: (i, 0)),
            _WBLK,
            _BBLK,
        ],
        out_specs=_QOUT,
        out_shape=_QSHAPE,
    )(x, w, b)


def _sage_z(agg_refs, cnt_ref, x_refs, wl_ref, bl_ref, wr_ref):
    inv = 1.0 / jnp.maximum(cnt_ref[...], 1.0)
    mean = jnp.concatenate([a[...] for a in agg_refs], axis=1) * inv
    xfull = jnp.concatenate([x[...] for x in x_refs], axis=1)
    return jnp.maximum(_dot(mean, wl_ref[...]) + bl_ref[...]
                       + _dot(xfull, wr_ref[...]), 0.0)


def _sage_body(a0, a1, a2, a3, cnt_ref, x0, x1, x2, x3,
               wl_ref, bl_ref, wr_ref, *out_refs):
    z = _sage_z((a0, a1, a2, a3), cnt_ref, (x0, x1, x2, x3),
                wl_ref, bl_ref, wr_ref)
    _split4(out_refs, z)


def _tc_sage(aggs, cnt, xs, wl, bl, wr):
    return pl.pallas_call(
        _sage_body,
        grid=(N // _RB,),
        in_specs=[_QBLK] * 4 + [pl.BlockSpec((_RB, 1), lambda i: (i, 0))]
                 + [_QBLK] * 4 + [_WBLK, _BBLK, _WBLK],
        out_specs=_QOUT,
        out_shape=_QSHAPE,
    )(*aggs, cnt, *xs, wl, bl, wr)


def _sage_dec_body(a0, a1, a2, a3, cnt_ref, x0, x1, x2, x3,
                   wl_ref, bl_ref, wr_ref, wd_ref, bd_ref, out_ref):
    z = _sage_z((a0, a1, a2, a3), cnt_ref, (x0, x1, x2, x3),
                wl_ref, bl_ref, wr_ref)
    out_ref[...] = _dot(z, wd_ref[...]) + bd_ref[...]


def _tc_sage_dec(aggs, cnt, xs, wl, bl, wr, wd, bd):
    return pl.pallas_call(
        _sage_dec_body,
        grid=(N // _RB,),
        in_specs=[_QBLK] * 4 + [pl.BlockSpec((_RB, 1), lambda i: (i, 0))]
                 + [_QBLK] * 4 + [_WBLK, _BBLK, _WBLK, _WBLK, _BBLK],
        out_specs=pl.BlockSpec((_RB, H), lambda i: (i, 0)),
        out_shape=jax.ShapeDtypeStruct((N, H), jnp.float32),
    )(*aggs, cnt, *xs, wl, bl, wr, wd, bd)


def kernel(x_drug, x_reaction, ei_drug_to_reaction, ei_reaction_rev_drug,
           edge_label_index,
           W_drug_lin, b_drug_lin, W_reaction_lin, b_reaction_lin,
           Wl1_dr, bl1_dr, Wr1_dr, Wl1_rd, bl1_rd, Wr1_rd,
           Wl2_dr, bl2_dr, Wr2_dr, Wl2_rd, bl2_rd, Wr2_rd,
           W_dec1, b_dec1, W_dec2, b_dec2):
    f32 = jnp.float32
    i32 = jnp.int32

    src_dr = ei_drug_to_reaction[0].astype(i32).reshape(NS, ANB, AEB)
    dst_dr = ei_drug_to_reaction[1].astype(i32).reshape(NS, ANB, AEB)
    src_rd = ei_reaction_rev_drug[0].astype(i32).reshape(NS, ANB, AEB)
    dst_rd = ei_reaction_rev_drug[1].astype(i32).reshape(NS, ANB, AEB)

    pad = jnp.zeros((LPAD - L,), i32)
    row3 = jnp.concatenate([edge_label_index[0].astype(i32), pad]
                           ).reshape(NC * NS, LPT)
    col3 = jnp.concatenate([edge_label_index[1].astype(i32), pad]
                           ).reshape(NC * NS, LPT)

    zeros_s = jnp.zeros((RCH, 16), f32)
    zeros_b = jnp.zeros((RCH, QC), f32)

    b_drug = b_drug_lin.reshape(1, H)
    b_react = b_reaction_lin.reshape(1, H)

    cnts = _sc_counts(dst_dr, dst_rd, zeros_s)
    cnt_r = cnts[0, :, 0:1]
    cnt_d = cnts[1, :, 0:1]

    xd = _tc_proj(x_drug, W_drug_lin, b_drug)
    xr = _tc_proj(x_reaction, W_reaction_lin, b_react)

    agg1 = _sc_agg(*xd, *xr, src_dr, dst_dr, src_rd, dst_rd, zeros_b)
    agg_r1, agg_d1 = agg1[:4], agg1[4:]
    zr = _tc_sage(agg_r1, cnt_r, xr, Wl1_dr, bl1_dr.reshape(1, H), Wr1_dr)
    zd = _tc_sage(agg_d1, cnt_d, xd, Wl1_rd, bl1_rd.reshape(1, H), Wr1_rd)

    agg2 = _sc_agg(*zd, *zr, src_dr, dst_dr, src_rd, dst_rd, zeros_b)
    agg_r2, agg_d2 = agg2[:4], agg2[4:]
    p_t = _tc_sage_dec(agg_d2, cnt_d, zd, Wl2_rd, bl2_rd.reshape(1, H),
                       Wr2_rd, W_dec1[:H], b_dec1.reshape(1, H))
    q_t = _tc_sage_dec(agg_r2, cnt_r, zr, Wl2_dr, bl2_dr.reshape(1, H),
                       Wr2_dr, W_dec1[H:], jnp.zeros((1, H), f32))

    w2 = W_dec2[:, 0]
    b2v = jnp.zeros((16,), f32).at[0].set(b_dec2[0])
    out = _sc_decoder(p_t, q_t, row3, col3, w2, b2v)
    return out[:L]


# speedup vs baseline: 1.1253x; 1.1253x over previous
import functools

import jax
import jax.numpy as jnp
from jax import lax
from jax.experimental import pallas as pl
from jax.experimental.pallas import tpu as pltpu
import jax.experimental.pallas.tpu_sc as plsc

N = 10000
H = 256
HC = 128
E = 160000
L = 40000
NC, NS = 2, 16
EB = 80
EPT = E // NS
AEB = 125
ANB = EPT // AEB
NBAT = EPT // EB
RCH = 80
NRCH = N // RCH
LPAD = 40960
LPT = LPAD // (NC * NS)
LBAT = LPT // EB

_MESH = plsc.VectorSubcoreMesh(core_axis_name="c", subcore_axis_name="s")
_UNTILED = pltpu.CompilerParams(use_tc_tiling_on_sc=False)


def _count_body(dst_a, dst_b, zeros_s, out, didx, ones_v, stage, hist):
    c = lax.axis_index("c")
    s = lax.axis_index("s")

    def fill_ones(i, _):
        ones_v[i, :] = jnp.ones((16,), jnp.float32)
        return 0

    lax.fori_loop(0, AEB, fill_ones, 0)

    @pl.when(c == 0)
    def _():
        pltpu.sync_copy(dst_a.at[s], didx)

    @pl.when(c == 1)
    def _():
        pltpu.sync_copy(dst_b.at[s], didx)

    nch = jnp.where(s < NRCH - 7 * NS, 8, 7)

    def zero(k, _):
        pltpu.sync_copy(zeros_s, hist.at[pl.ds((s + k * NS) * RCH, RCH)])
        return 0

    lax.fori_loop(0, nch, zero, 0)
    plsc.subcore_barrier()

    def scat(i, _):
        pltpu.sync_copy(ones_v, hist.at[didx.at[i]], add=True)
        return 0

    lax.fori_loop(0, ANB, scat, 0)
    plsc.subcore_barrier()

    def wb(k, _):
        off = (s + k * NS) * RCH
        pltpu.sync_copy(hist.at[pl.ds(off, RCH)], stage)
        pltpu.sync_copy(stage, out.at[c, pl.ds(off, RCH)])
        return 0

    lax.fori_loop(0, nch, wb, 0)


@functools.partial(
    pl.kernel,
    out_type=jax.ShapeDtypeStruct((NC, N, 16), jnp.float32),
    mesh=_MESH,
    compiler_params=_UNTILED,
    scratch_types=[
        pltpu.VMEM((ANB, AEB), jnp.int32),
        pltpu.VMEM((AEB, 16), jnp.float32),
        pltpu.VMEM((RCH, 16), jnp.float32),
        pltpu.VMEM_SHARED((N, 16), jnp.float32),
    ],
)
def _sc_counts(dst_a, dst_b, zeros_s, out, didx, ones_v, stage, hist):
    _count_body(dst_a, dst_b, zeros_s, out, didx, ones_v, stage, hist)


QC = 64


def _agg_body(x0, x1, x2, x3, src3, dst3, zeros_b, out0, out1, out2, out3,
              sidx, didx, *rest):
    bufs = rest[0:8]
    zb = rest[8]
    acc = rest[9]
    gsems = rest[10:18]
    ssems = rest[18:26]
    nb = len(bufs)
    c = lax.axis_index("c")
    s = lax.axis_index("s")

    pltpu.sync_copy(src3.at[s], sidx)
    pltpu.sync_copy(dst3.at[s], didx)

    nch = jnp.where(s < NRCH - 7 * NS, 8, 7)

    def zero(k, _):
        pltpu.sync_copy(zeros_b, acc.at[pl.ds((s + k * NS) * RCH, RCH)])
        return 0

    for p, (xa, xb, oa, ob) in enumerate(((x0, x2, out0, out2),
                                          (x1, x3, out1, out3))):
        lax.fori_loop(0, nch, zero, 0)
        plsc.subcore_barrier()

        def start_gather(i, buf, gsem):
            @pl.when(c == 0)
            def _():
                pltpu.async_copy(xa.at[sidx.at[i]], buf, gsem)

            @pl.when(c == 1)
            def _():
                pltpu.async_copy(xb.at[sidx.at[i]], buf, gsem)

        def wait_gather(buf, gsem):
            pltpu.make_async_copy(xa.at[sidx.at[0]], buf, gsem).wait()

        def start_scatter(i, buf, ssem):
            pltpu.async_copy(buf, acc.at[didx.at[i]], ssem, add=True)

        def wait_scatter(i, buf, ssem):
            pltpu.make_async_copy(buf, acc.at[didx.at[i]], ssem).wait()

        for b in range(nb):
            start_gather(b, bufs[b], gsems[b])

        def bodyn(k, _):
            i0 = nb * k
            for b in range(nb):
                wait_gather(bufs[b], gsems[b])
                start_scatter(i0 + b, bufs[b], ssems[b])
            for b in range(nb):
                wait_scatter(i0 + b, bufs[b], ssems[b])

                @pl.when(i0 + b + nb < ANB)
                def _():
                    start_gather(i0 + b + nb, bufs[b], gsems[b])

            return 0

        lax.fori_loop(0, ANB // nb, bodyn, 0)
        plsc.subcore_barrier()

        def wb(k, _):
            off = (s + k * NS) * RCH
            pltpu.sync_copy(acc.at[pl.ds(off, RCH)], zb)

            @pl.when(c == 0)
            def _():
                pltpu.sync_copy(zb, oa.at[pl.ds(off, RCH)])

            @pl.when(c == 1)
            def _():
                pltpu.sync_copy(zb, ob.at[pl.ds(off, RCH)])

            return 0

        lax.fori_loop(0, nch, wb, 0)
        if p == 0:
            plsc.subcore_barrier()


_QSDS = jax.ShapeDtypeStruct((N, QC), jnp.float32)


@functools.partial(
    pl.kernel,
    out_type=(_QSDS,) * 4,
    mesh=_MESH,
    compiler_params=_UNTILED,
    scratch_types=[
        pltpu.VMEM((ANB, AEB), jnp.int32),
        pltpu.VMEM((ANB, AEB), jnp.int32),
    ] + [pltpu.VMEM((AEB, QC), jnp.float32)] * 8 + [
        pltpu.VMEM((RCH, QC), jnp.float32),
        pltpu.VMEM_SHARED((N, QC), jnp.float32),
    ] + [pltpu.SemaphoreType.DMA] * 16,
)
def _sc_agg(*args):
    _agg_body(*args)


def _dec_body(p_t, q_t, row3, col3, w2, b2v, out,
              ridx, cidx, w2v, b2s, pr0, qr0, pr1, qr1, outv,
              semp0, semq0, semp1, semq1):
    c = lax.axis_index("c")
    s = lax.axis_index("s")
    wid = s * NC + c

    pltpu.sync_copy(row3.at[wid], ridx)
    pltpu.sync_copy(col3.at[wid], cidx)
    pltpu.sync_copy(w2, w2v)
    pltpu.sync_copy(b2v, b2s)

    prs = (pr0, pr1)
    qrs = (qr0, qr1)
    psems = (semp0, semp1)
    qsems = (semq0, semq1)

    def start(j, b):
        pltpu.async_copy(p_t.at[ridx.at[pl.ds(j * EB, EB)]], prs[b], psems[b])
        pltpu.async_copy(q_t.at[cidx.at[pl.ds(j * EB, EB)]], qrs[b], qsems[b])

    def wait(b):
        pltpu.make_async_copy(p_t.at[ridx.at[pl.ds(0, EB)]],
                              prs[b], psems[b]).wait()
        pltpu.make_async_copy(q_t.at[cidx.at[pl.ds(0, EB)]],
                              qrs[b], qsems[b]).wait()

    lanes = lax.iota(jnp.int32, 16)

    def compute(j, pr, qr):
        def edge(e, vec):
            acc = b2s[...]
            for h in range(H // 16):
                pch = pr[e, pl.ds(h * 16, 16)]
                qch = qr[e, pl.ds(h * 16, 16)]
                g = jnp.maximum(pch + qch, 0.0)
                acc = acc + g * w2v[pl.ds(h * 16, 16)]
            lane = lax.rem(e, 16)
            vec = jnp.where(lanes == lane, jnp.sum(acc), vec)

            @pl.when(lane == 15)
            def _():
                outv[pl.ds(j * EB + e - 15, 16)] = vec

            return vec

        lax.fori_loop(0, EB, edge, jnp.zeros((16,), jnp.float32))

    start(0, 0)

    def pair(k, _):
        j0 = 2 * k
        start(j0 + 1, 1)
        wait(0)
        compute(j0, pr0, qr0)

        @pl.when(j0 + 2 < LBAT)
        def _():
            start(j0 + 2, 0)

        wait(1)
        compute(j0 + 1, pr1, qr1)
        return 0

    lax.fori_loop(0, LBAT // 2, pair, 0)
    pltpu.sync_copy(outv, out.at[pl.ds(wid * LPT, LPT)])


@functools.partial(
    pl.kernel,
    out_type=jax.ShapeDtypeStruct((LPAD,), jnp.float32),
    mesh=_MESH,
    compiler_params=pltpu.CompilerParams(needs_layout_passes=False),
    scratch_types=[
        pltpu.VMEM((LPT,), jnp.int32),
        pltpu.VMEM((LPT,), jnp.int32),
        pltpu.VMEM((H,), jnp.float32),
        pltpu.VMEM((16,), jnp.float32),
        pltpu.VMEM((EB, H), jnp.float32),
        pltpu.VMEM((EB, H), jnp.float32),
        pltpu.VMEM((EB, H), jnp.float32),
        pltpu.VMEM((EB, H), jnp.float32),
        pltpu.VMEM((LPT,), jnp.float32),
        pltpu.SemaphoreType.DMA,
        pltpu.SemaphoreType.DMA,
        pltpu.SemaphoreType.DMA,
        pltpu.SemaphoreType.DMA,
    ],
)
def _sc_decoder(p_t, q_t, row3, col3, w2, b2v, out,
                ridx, cidx, w2v, b2s, pr0, qr0, pr1, qr1, outv,
                semp0, semq0, semp1, semq1):
    _dec_body(p_t, q_t, row3, col3, w2, b2v, out,
              ridx, cidx, w2v, b2s, pr0, qr0, pr1, qr1, outv,
              semp0, semq0, semp1, semq1)


_RB = 1000


def _dot(a, b):
    return jnp.dot(a, b, preferred_element_type=jnp.float32)


_QBLK = pl.BlockSpec((_RB, QC), lambda i: (i, 0))
_WBLK = pl.BlockSpec((H, H), lambda i: (0, 0))
_BBLK = pl.BlockSpec((1, H), lambda i: (0, 0))
_QOUT = (_QBLK, _QBLK, _QBLK, _QBLK)
_QSHAPE = tuple(jax.ShapeDtypeStruct((N, QC), jnp.float32) for _ in range(4))


def _split4(out_refs, y):
    for k in range(4):
        out_refs[k][...] = y[:, k * QC:(k + 1) * QC]


def _proj_body(x_ref, w_ref, b_ref, *out_refs):
    _split4(out_refs, _dot(x_ref[...], w_ref[...]) + b_ref[...])


def _tc_proj(x, w, b):
    return pl.pallas_call(
        _proj_body,
        grid=(N // _RB,),
        in_specs=[
            pl.BlockSpec((_RB, H), lambda i: (i, 0)),
            _WBLK,
            _BBLK,
        ],
        out_specs=_QOUT,
        out_shape=_QSHAPE,
    )(x, w, b)


def _sage_z(agg_refs, cnt_ref, x_refs, wl_ref, bl_ref, wr_ref):
    inv = 1.0 / jnp.maximum(cnt_ref[...], 1.0)
    mean = jnp.concatenate([a[...] for a in agg_refs], axis=1) * inv
    xfull = jnp.concatenate([x[...] for x in x_refs], axis=1)
    return jnp.maximum(_dot(mean, wl_ref[...]) + bl_ref[...]
                       + _dot(xfull, wr_ref[...]), 0.0)


def _sage_body(a0, a1, a2, a3, cnt_ref, x0, x1, x2, x3,
               wl_ref, bl_ref, wr_ref, *out_refs):
    z = _sage_z((a0, a1, a2, a3), cnt_ref, (x0, x1, x2, x3),
                wl_ref, bl_ref, wr_ref)
    _split4(out_refs, z)


def _tc_sage(aggs, cnt, xs, wl, bl, wr):
    return pl.pallas_call(
        _sage_body,
        grid=(N // _RB,),
        in_specs=[_QBLK] * 4 + [pl.BlockSpec((_RB, 1), lambda i: (i, 0))]
                 + [_QBLK] * 4 + [_WBLK, _BBLK, _WBLK],
        out_specs=_QOUT,
        out_shape=_QSHAPE,
    )(*aggs, cnt, *xs, wl, bl, wr)


def _sage_dec_body(a0, a1, a2, a3, cnt_ref, x0, x1, x2, x3,
                   wl_ref, bl_ref, wr_ref, wd_ref, bd_ref, out_ref):
    z = _sage_z((a0, a1, a2, a3), cnt_ref, (x0, x1, x2, x3),
                wl_ref, bl_ref, wr_ref)
    out_ref[...] = _dot(z, wd_ref[...]) + bd_ref[...]


def _tc_sage_dec(aggs, cnt, xs, wl, bl, wr, wd, bd):
    return pl.pallas_call(
        _sage_dec_body,
        grid=(N // _RB,),
        in_specs=[_QBLK] * 4 + [pl.BlockSpec((_RB, 1), lambda i: (i, 0))]
                 + [_QBLK] * 4 + [_WBLK, _BBLK, _WBLK, _WBLK, _BBLK],
        out_specs=pl.BlockSpec((_RB, H), lambda i: (i, 0)),
        out_shape=jax.ShapeDtypeStruct((N, H), jnp.float32),
    )(*aggs, cnt, *xs, wl, bl, wr, wd, bd)


def kernel(x_drug, x_reaction, ei_drug_to_reaction, ei_reaction_rev_drug,
           edge_label_index,
           W_drug_lin, b_drug_lin, W_reaction_lin, b_reaction_lin,
           Wl1_dr, bl1_dr, Wr1_dr, Wl1_rd, bl1_rd, Wr1_rd,
           Wl2_dr, bl2_dr, Wr2_dr, Wl2_rd, bl2_rd, Wr2_rd,
           W_dec1, b_dec1, W_dec2, b_dec2):
    f32 = jnp.float32
    i32 = jnp.int32

    src_dr = ei_drug_to_reaction[0].astype(i32).reshape(NS, ANB, AEB)
    dst_dr = ei_drug_to_reaction[1].astype(i32).reshape(NS, ANB, AEB)
    src_rd = ei_reaction_rev_drug[0].astype(i32).reshape(NS, ANB, AEB)
    dst_rd = ei_reaction_rev_drug[1].astype(i32).reshape(NS, ANB, AEB)

    pad = jnp.zeros((LPAD - L,), i32)
    row3 = jnp.concatenate([edge_label_index[0].astype(i32), pad]
                           ).reshape(NC * NS, LPT)
    col3 = jnp.concatenate([edge_label_index[1].astype(i32), pad]
                           ).reshape(NC * NS, LPT)

    zeros_s = jnp.zeros((RCH, 16), f32)
    zeros_b = jnp.zeros((RCH, QC), f32)

    b_drug = b_drug_lin.reshape(1, H)
    b_react = b_reaction_lin.reshape(1, H)

    cnts = _sc_counts(dst_dr, dst_rd, zeros_s)
    cnt_r = cnts[0, :, 0:1]
    cnt_d = cnts[1, :, 0:1]

    xd = _tc_proj(x_drug, W_drug_lin, b_drug)
    xr = _tc_proj(x_reaction, W_reaction_lin, b_react)

    agg_r1 = _sc_agg(*xd, src_dr, dst_dr, zeros_b)
    agg_d1 = _sc_agg(*xr, src_rd, dst_rd, zeros_b)
    zr = _tc_sage(agg_r1, cnt_r, xr, Wl1_dr, bl1_dr.reshape(1, H), Wr1_dr)
    zd = _tc_sage(agg_d1, cnt_d, xd, Wl1_rd, bl1_rd.reshape(1, H), Wr1_rd)

    agg_r2 = _sc_agg(*zd, src_dr, dst_dr, zeros_b)
    agg_d2 = _sc_agg(*zr, src_rd, dst_rd, zeros_b)
    p_t = _tc_sage_dec(agg_d2, cnt_d, zd, Wl2_rd, bl2_rd.reshape(1, H),
                       Wr2_rd, W_dec1[:H], b_dec1.reshape(1, H))
    q_t = _tc_sage_dec(agg_r2, cnt_r, zr, Wl2_dr, bl2_dr.reshape(1, H),
                       Wr2_dr, W_dec1[H:], jnp.zeros((1, H), f32))

    w2 = W_dec2[:, 0]
    b2v = jnp.zeros((16,), f32).at[0].set(b_dec2[0])
    out = _sc_decoder(p_t, q_t, row3, col3, w2, b2v)
    return out[:L]


# speedup vs baseline: 1.2696x; 1.1282x over previous
import functools

import jax
import jax.numpy as jnp
from jax import lax
from jax.experimental import pallas as pl
from jax.experimental.pallas import tpu as pltpu
import jax.experimental.pallas.tpu_sc as plsc

N = 10000
H = 256
HC = 128
E = 160000
L = 40000
NC, NS = 2, 16
EB = 80
EPT = E // NS
AEB = 125
ANB = EPT // AEB
NBAT = EPT // EB
RCH = 80
NRCH = N // RCH
LPAD = 40960
LPT = LPAD // (NC * NS)
LBAT = LPT // EB

_MESH = plsc.VectorSubcoreMesh(core_axis_name="c", subcore_axis_name="s")
_UNTILED = pltpu.CompilerParams(use_tc_tiling_on_sc=False)


def _count_body(dst_a, dst_b, zeros_s, out, didx, ones_v, stage, hist):
    c = lax.axis_index("c")
    s = lax.axis_index("s")

    def fill_ones(i, _):
        ones_v[i, :] = jnp.ones((16,), jnp.float32)
        return 0

    lax.fori_loop(0, AEB, fill_ones, 0)

    @pl.when(c == 0)
    def _():
        pltpu.sync_copy(dst_a.at[s], didx)

    @pl.when(c == 1)
    def _():
        pltpu.sync_copy(dst_b.at[s], didx)

    nch = jnp.where(s < NRCH - 7 * NS, 8, 7)

    def zero(k, _):
        pltpu.sync_copy(zeros_s, hist.at[pl.ds((s + k * NS) * RCH, RCH)])
        return 0

    lax.fori_loop(0, nch, zero, 0)
    plsc.subcore_barrier()

    def scat(i, _):
        pltpu.sync_copy(ones_v, hist.at[didx.at[i]], add=True)
        return 0

    lax.fori_loop(0, ANB, scat, 0)
    plsc.subcore_barrier()

    def wb(k, _):
        off = (s + k * NS) * RCH
        pltpu.sync_copy(hist.at[pl.ds(off, RCH)], stage)
        pltpu.sync_copy(stage, out.at[c, pl.ds(off, RCH)])
        return 0

    lax.fori_loop(0, nch, wb, 0)


@functools.partial(
    pl.kernel,
    out_type=jax.ShapeDtypeStruct((NC, N, 16), jnp.float32),
    mesh=_MESH,
    compiler_params=_UNTILED,
    scratch_types=[
        pltpu.VMEM((ANB, AEB), jnp.int32),
        pltpu.VMEM((AEB, 16), jnp.float32),
        pltpu.VMEM((RCH, 16), jnp.float32),
        pltpu.VMEM_SHARED((N, 16), jnp.float32),
    ],
)
def _sc_counts(dst_a, dst_b, zeros_s, out, didx, ones_v, stage, hist):
    _count_body(dst_a, dst_b, zeros_s, out, didx, ones_v, stage, hist)


QC = 64


def _agg_body(x0, x1, x2, x3, src3, dst3, zeros_b, out0, out1, out2, out3,
              sidx, didx, *rest):
    bufs = rest[0:8]
    acc = rest[8]
    gsems = rest[9:17]
    ssems = rest[17:25]
    nb = len(bufs)
    c = lax.axis_index("c")
    s = lax.axis_index("s")

    pltpu.sync_copy(src3.at[s], sidx)
    pltpu.sync_copy(dst3.at[s], didx)

    def zero_acc():
        @pl.when(s < NS - 1)
        def _():
            pltpu.sync_copy(zeros_b, acc.at[pl.ds(s * 640, 640)])

        @pl.when(s == NS - 1)
        def _():
            pltpu.sync_copy(zeros_b.at[pl.ds(0, 400)],
                            acc.at[pl.ds(9600, 400)])

    for p, (xa, xb, oa, ob) in enumerate(((x0, x2, out0, out2),
                                          (x1, x3, out1, out3))):
        zero_acc()
        plsc.subcore_barrier()

        def start_gather(i, buf, gsem):
            @pl.when(c == 0)
            def _():
                pltpu.async_copy(xa.at[sidx.at[i]], buf, gsem)

            @pl.when(c == 1)
            def _():
                pltpu.async_copy(xb.at[sidx.at[i]], buf, gsem)

        def wait_gather(buf, gsem):
            pltpu.make_async_copy(xa.at[sidx.at[0]], buf, gsem).wait()

        def start_scatter(i, buf, ssem):
            pltpu.async_copy(buf, acc.at[didx.at[i]], ssem, add=True)

        def wait_scatter(i, buf, ssem):
            pltpu.make_async_copy(buf, acc.at[didx.at[i]], ssem).wait()

        for b in range(nb):
            start_gather(b, bufs[b], gsems[b])

        def bodyn(k, _):
            i0 = nb * k
            for b in range(nb):
                wait_gather(bufs[b], gsems[b])
                start_scatter(i0 + b, bufs[b], ssems[b])
            for b in range(nb):
                wait_scatter(i0 + b, bufs[b], ssems[b])

                @pl.when(i0 + b + nb < ANB)
                def _():
                    start_gather(i0 + b + nb, bufs[b], gsems[b])

            return 0

        lax.fori_loop(0, ANB // nb, bodyn, 0)
        plsc.subcore_barrier()

        for cc, ox in ((0, oa), (1, ob)):
            @pl.when((c == cc) & (s < NS - 1))
            def _():
                pltpu.sync_copy(acc.at[pl.ds(s * 640, 640)],
                                ox.at[pl.ds(s * 640, 640)])

            @pl.when((c == cc) & (s == NS - 1))
            def _():
                pltpu.sync_copy(acc.at[pl.ds(9600, 400)],
                                ox.at[pl.ds(9600, 400)])

        if p == 0:
            plsc.subcore_barrier()


_QSDS = jax.ShapeDtypeStruct((N, QC), jnp.float32)


@functools.partial(
    pl.kernel,
    out_type=(_QSDS,) * 4,
    mesh=_MESH,
    compiler_params=_UNTILED,
    scratch_types=[
        pltpu.VMEM((ANB, AEB), jnp.int32),
        pltpu.VMEM((ANB, AEB), jnp.int32),
    ] + [pltpu.VMEM((AEB, QC), jnp.float32)] * 8 + [
        pltpu.VMEM_SHARED((N, QC), jnp.float32),
    ] + [pltpu.SemaphoreType.DMA] * 16,
)
def _sc_agg(*args):
    _agg_body(*args)


def _dec_body(p_t, q_t, row3, col3, w2, b2v, out,
              ridx, cidx, w2v, b2s, pr0, qr0, pr1, qr1, outv,
              semp0, semq0, semp1, semq1):
    c = lax.axis_index("c")
    s = lax.axis_index("s")
    wid = s * NC + c

    pltpu.sync_copy(row3.at[wid], ridx)
    pltpu.sync_copy(col3.at[wid], cidx)
    pltpu.sync_copy(w2, w2v)
    pltpu.sync_copy(b2v, b2s)

    prs = (pr0, pr1)
    qrs = (qr0, qr1)
    psems = (semp0, semp1)
    qsems = (semq0, semq1)

    def start(j, b):
        pltpu.async_copy(p_t.at[ridx.at[pl.ds(j * EB, EB)]], prs[b], psems[b])
        pltpu.async_copy(q_t.at[cidx.at[pl.ds(j * EB, EB)]], qrs[b], qsems[b])

    def wait(b):
        pltpu.make_async_copy(p_t.at[ridx.at[pl.ds(0, EB)]],
                              prs[b], psems[b]).wait()
        pltpu.make_async_copy(q_t.at[cidx.at[pl.ds(0, EB)]],
                              qrs[b], qsems[b]).wait()

    lanes = lax.iota(jnp.int32, 16)

    def compute(j, pr, qr):
        def edge(e, vec):
            acc = b2s[...]
            for h in range(H // 16):
                pch = pr[e, pl.ds(h * 16, 16)]
                qch = qr[e, pl.ds(h * 16, 16)]
                g = jnp.maximum(pch + qch, 0.0)
                acc = acc + g * w2v[pl.ds(h * 16, 16)]
            lane = lax.rem(e, 16)
            vec = jnp.where(lanes == lane, jnp.sum(acc), vec)

            @pl.when(lane == 15)
            def _():
                outv[pl.ds(j * EB + e - 15, 16)] = vec

            return vec

        lax.fori_loop(0, EB, edge, jnp.zeros((16,), jnp.float32))

    start(0, 0)

    def pair(k, _):
        j0 = 2 * k
        start(j0 + 1, 1)
        wait(0)
        compute(j0, pr0, qr0)

        @pl.when(j0 + 2 < LBAT)
        def _():
            start(j0 + 2, 0)

        wait(1)
        compute(j0 + 1, pr1, qr1)
        return 0

    lax.fori_loop(0, LBAT // 2, pair, 0)
    pltpu.sync_copy(outv, out.at[pl.ds(wid * LPT, LPT)])


@functools.partial(
    pl.kernel,
    out_type=jax.ShapeDtypeStruct((LPAD,), jnp.float32),
    mesh=_MESH,
    compiler_params=pltpu.CompilerParams(needs_layout_passes=False),
    scratch_types=[
        pltpu.VMEM((LPT,), jnp.int32),
        pltpu.VMEM((LPT,), jnp.int32),
        pltpu.VMEM((H,), jnp.float32),
        pltpu.VMEM((16,), jnp.float32),
        pltpu.VMEM((EB, H), jnp.float32),
        pltpu.VMEM((EB, H), jnp.float32),
        pltpu.VMEM((EB, H), jnp.float32),
        pltpu.VMEM((EB, H), jnp.float32),
        pltpu.VMEM((LPT,), jnp.float32),
        pltpu.SemaphoreType.DMA,
        pltpu.SemaphoreType.DMA,
        pltpu.SemaphoreType.DMA,
        pltpu.SemaphoreType.DMA,
    ],
)
def _sc_decoder(p_t, q_t, row3, col3, w2, b2v, out,
                ridx, cidx, w2v, b2s, pr0, qr0, pr1, qr1, outv,
                semp0, semq0, semp1, semq1):
    _dec_body(p_t, q_t, row3, col3, w2, b2v, out,
              ridx, cidx, w2v, b2s, pr0, qr0, pr1, qr1, outv,
              semp0, semq0, semp1, semq1)


_RB = 1000


def _dot(a, b):
    return jnp.dot(a, b, preferred_element_type=jnp.float32)


_QBLK = pl.BlockSpec((_RB, QC), lambda i: (i, 0))
_WBLK = pl.BlockSpec((H, H), lambda i: (0, 0))
_BBLK = pl.BlockSpec((1, H), lambda i: (0, 0))
_QOUT = (_QBLK, _QBLK, _QBLK, _QBLK)
_QSHAPE = tuple(jax.ShapeDtypeStruct((N, QC), jnp.float32) for _ in range(4))


def _split4(out_refs, y):
    for k in range(4):
        out_refs[k][...] = y[:, k * QC:(k + 1) * QC]


def _proj_body(x_ref, w_ref, b_ref, *out_refs):
    _split4(out_refs, _dot(x_ref[...], w_ref[...]) + b_ref[...])


def _tc_proj(x, w, b):
    return pl.pallas_call(
        _proj_body,
        grid=(N // _RB,),
        in_specs=[
            pl.BlockSpec((_RB, H), lambda i: (i, 0)),
            _WBLK,
            _BBLK,
        ],
        out_specs=_QOUT,
        out_shape=_QSHAPE,
    )(x, w, b)


def _sage_z(agg_refs, cnt_ref, x_refs, wl_ref, bl_ref, wr_ref):
    inv = 1.0 / jnp.maximum(cnt_ref[...], 1.0)
    mean = jnp.concatenate([a[...] for a in agg_refs], axis=1) * inv
    xfull = jnp.concatenate([x[...] for x in x_refs], axis=1)
    return jnp.maximum(_dot(mean, wl_ref[...]) + bl_ref[...]
                       + _dot(xfull, wr_ref[...]), 0.0)


def _sage_body(a0, a1, a2, a3, cnt_ref, x0, x1, x2, x3,
               wl_ref, bl_ref, wr_ref, *out_refs):
    z = _sage_z((a0, a1, a2, a3), cnt_ref, (x0, x1, x2, x3),
                wl_ref, bl_ref, wr_ref)
    _split4(out_refs, z)


def _tc_sage(aggs, cnt, xs, wl, bl, wr):
    return pl.pallas_call(
        _sage_body,
        grid=(N // _RB,),
        in_specs=[_QBLK] * 4 + [pl.BlockSpec((_RB, 1), lambda i: (i, 0))]
                 + [_QBLK] * 4 + [_WBLK, _BBLK, _WBLK],
        out_specs=_QOUT,
        out_shape=_QSHAPE,
    )(*aggs, cnt, *xs, wl, bl, wr)


def _sage_dec_body(a0, a1, a2, a3, cnt_ref, x0, x1, x2, x3,
                   wl_ref, bl_ref, wr_ref, wd_ref, bd_ref, out_ref):
    z = _sage_z((a0, a1, a2, a3), cnt_ref, (x0, x1, x2, x3),
                wl_ref, bl_ref, wr_ref)
    out_ref[...] = _dot(z, wd_ref[...]) + bd_ref[...]


def _tc_sage_dec(aggs, cnt, xs, wl, bl, wr, wd, bd):
    return pl.pallas_call(
        _sage_dec_body,
        grid=(N // _RB,),
        in_specs=[_QBLK] * 4 + [pl.BlockSpec((_RB, 1), lambda i: (i, 0))]
                 + [_QBLK] * 4 + [_WBLK, _BBLK, _WBLK, _WBLK, _BBLK],
        out_specs=pl.BlockSpec((_RB, H), lambda i: (i, 0)),
        out_shape=jax.ShapeDtypeStruct((N, H), jnp.float32),
    )(*aggs, cnt, *xs, wl, bl, wr, wd, bd)


def kernel(x_drug, x_reaction, ei_drug_to_reaction, ei_reaction_rev_drug,
           edge_label_index,
           W_drug_lin, b_drug_lin, W_reaction_lin, b_reaction_lin,
           Wl1_dr, bl1_dr, Wr1_dr, Wl1_rd, bl1_rd, Wr1_rd,
           Wl2_dr, bl2_dr, Wr2_dr, Wl2_rd, bl2_rd, Wr2_rd,
           W_dec1, b_dec1, W_dec2, b_dec2):
    f32 = jnp.float32
    i32 = jnp.int32

    src_dr = ei_drug_to_reaction[0].astype(i32).reshape(NS, ANB, AEB)
    dst_dr = ei_drug_to_reaction[1].astype(i32).reshape(NS, ANB, AEB)
    src_rd = ei_reaction_rev_drug[0].astype(i32).reshape(NS, ANB, AEB)
    dst_rd = ei_reaction_rev_drug[1].astype(i32).reshape(NS, ANB, AEB)

    pad = jnp.zeros((LPAD - L,), i32)
    row3 = jnp.concatenate([edge_label_index[0].astype(i32), pad]
                           ).reshape(NC * NS, LPT)
    col3 = jnp.concatenate([edge_label_index[1].astype(i32), pad]
                           ).reshape(NC * NS, LPT)

    zeros_s = jnp.zeros((RCH, 16), f32)
    zeros_b = jnp.zeros((640, QC), f32)

    b_drug = b_drug_lin.reshape(1, H)
    b_react = b_reaction_lin.reshape(1, H)

    cnts = _sc_counts(dst_dr, dst_rd, zeros_s)
    cnt_r = cnts[0, :, 0:1]
    cnt_d = cnts[1, :, 0:1]

    xd = _tc_proj(x_drug, W_drug_lin, b_drug)
    xr = _tc_proj(x_reaction, W_reaction_lin, b_react)

    agg_r1 = _sc_agg(*xd, src_dr, dst_dr, zeros_b)
    agg_d1 = _sc_agg(*xr, src_rd, dst_rd, zeros_b)
    zr = _tc_sage(agg_r1, cnt_r, xr, Wl1_dr, bl1_dr.reshape(1, H), Wr1_dr)
    zd = _tc_sage(agg_d1, cnt_d, xd, Wl1_rd, bl1_rd.reshape(1, H), Wr1_rd)

    agg_r2 = _sc_agg(*zd, src_dr, dst_dr, zeros_b)
    agg_d2 = _sc_agg(*zr, src_rd, dst_rd, zeros_b)
    p_t = _tc_sage_dec(agg_d2, cnt_d, zd, Wl2_rd, bl2_rd.reshape(1, H),
                       Wr2_rd, W_dec1[:H], b_dec1.reshape(1, H))
    q_t = _tc_sage_dec(agg_r2, cnt_r, zr, Wl2_dr, bl2_dr.reshape(1, H),
                       Wr2_dr, W_dec1[H:], jnp.zeros((1, H), f32))

    w2 = W_dec2[:, 0]
    b2v = jnp.zeros((16,), f32).at[0].set(b_dec2[0])
    out = _sc_decoder(p_t, q_t, row3, col3, w2, b2v)
    return out[:L]


# speedup vs baseline: 1.2953x; 1.0202x over previous
import functools

import jax
import jax.numpy as jnp
from jax import lax
from jax.experimental import pallas as pl
from jax.experimental.pallas import tpu as pltpu
import jax.experimental.pallas.tpu_sc as plsc

N = 10000
H = 256
HC = 128
E = 160000
L = 40000
NC, NS = 2, 16
EB = 80
EPT = E // NS
AEB = 125
ANB = EPT // AEB
NBAT = EPT // EB
RCH = 80
NRCH = N // RCH
LPAD = 40960
LPT = LPAD // (NC * NS)
LBAT = LPT // EB

_MESH = plsc.VectorSubcoreMesh(core_axis_name="c", subcore_axis_name="s")
_UNTILED = pltpu.CompilerParams(use_tc_tiling_on_sc=False)


def _count_body(dst_a, dst_b, zeros_s, out, didx, ones_v, hist):
    c = lax.axis_index("c")
    s = lax.axis_index("s")

    def fill_ones(i, _):
        ones_v[i, :] = jnp.ones((16,), jnp.float32)
        return 0

    lax.fori_loop(0, AEB, fill_ones, 0)

    @pl.when(c == 0)
    def _():
        pltpu.sync_copy(dst_a.at[s], didx)

    @pl.when(c == 1)
    def _():
        pltpu.sync_copy(dst_b.at[s], didx)

    @pl.when(s < NS - 1)
    def _():
        pltpu.sync_copy(zeros_s, hist.at[pl.ds(s * 640, 640)])

    @pl.when(s == NS - 1)
    def _():
        pltpu.sync_copy(zeros_s.at[pl.ds(0, 400)], hist.at[pl.ds(9600, 400)])

    plsc.subcore_barrier()

    def scat(i, _):
        pltpu.sync_copy(ones_v, hist.at[didx.at[i]], add=True)
        return 0

    lax.fori_loop(0, ANB, scat, 0)
    plsc.subcore_barrier()

    @pl.when(s < NS - 1)
    def _():
        pltpu.sync_copy(hist.at[pl.ds(s * 640, 640)],
                        out.at[c, pl.ds(s * 640, 640)])

    @pl.when(s == NS - 1)
    def _():
        pltpu.sync_copy(hist.at[pl.ds(9600, 400)],
                        out.at[c, pl.ds(9600, 400)])


@functools.partial(
    pl.kernel,
    out_type=jax.ShapeDtypeStruct((NC, N, 16), jnp.float32),
    mesh=_MESH,
    compiler_params=_UNTILED,
    scratch_types=[
        pltpu.VMEM((ANB, AEB), jnp.int32),
        pltpu.VMEM((AEB, 16), jnp.float32),
        pltpu.VMEM_SHARED((N, 16), jnp.float32),
    ],
)
def _sc_counts(dst_a, dst_b, zeros_s, out, didx, ones_v, hist):
    _count_body(dst_a, dst_b, zeros_s, out, didx, ones_v, hist)


QC = 64


def _agg_body(x0, x1, x2, x3, src3, dst3, zeros_b, out0, out1, out2, out3,
              sidx, didx, *rest):
    bufs = rest[0:8]
    acc = rest[8]
    gsems = rest[9:17]
    ssems = rest[17:25]
    nb = len(bufs)
    c = lax.axis_index("c")
    s = lax.axis_index("s")

    pltpu.sync_copy(src3.at[s], sidx)
    pltpu.sync_copy(dst3.at[s], didx)

    def zero_acc():
        @pl.when(s < NS - 1)
        def _():
            pltpu.sync_copy(zeros_b, acc.at[pl.ds(s * 640, 640)])

        @pl.when(s == NS - 1)
        def _():
            pltpu.sync_copy(zeros_b.at[pl.ds(0, 400)],
                            acc.at[pl.ds(9600, 400)])

    for p, (xa, xb, oa, ob) in enumerate(((x0, x2, out0, out2),
                                          (x1, x3, out1, out3))):
        zero_acc()
        plsc.subcore_barrier()

        def start_gather(i, buf, gsem):
            @pl.when(c == 0)
            def _():
                pltpu.async_copy(xa.at[sidx.at[i]], buf, gsem)

            @pl.when(c == 1)
            def _():
                pltpu.async_copy(xb.at[sidx.at[i]], buf, gsem)

        def wait_gather(buf, gsem):
            pltpu.make_async_copy(xa.at[sidx.at[0]], buf, gsem).wait()

        def start_scatter(i, buf, ssem):
            pltpu.async_copy(buf, acc.at[didx.at[i]], ssem, add=True)

        def wait_scatter(i, buf, ssem):
            pltpu.make_async_copy(buf, acc.at[didx.at[i]], ssem).wait()

        for b in range(nb):
            start_gather(b, bufs[b], gsems[b])

        def bodyn(k, _):
            i0 = nb * k
            for b in range(nb):
                wait_gather(bufs[b], gsems[b])
                start_scatter(i0 + b, bufs[b], ssems[b])
            for b in range(nb):
                wait_scatter(i0 + b, bufs[b], ssems[b])

                @pl.when(i0 + b + nb < ANB)
                def _():
                    start_gather(i0 + b + nb, bufs[b], gsems[b])

            return 0

        lax.fori_loop(0, ANB // nb, bodyn, 0)
        plsc.subcore_barrier()

        for cc, ox in ((0, oa), (1, ob)):
            @pl.when((c == cc) & (s < NS - 1))
            def _():
                pltpu.sync_copy(acc.at[pl.ds(s * 640, 640)],
                                ox.at[pl.ds(s * 640, 640)])

            @pl.when((c == cc) & (s == NS - 1))
            def _():
                pltpu.sync_copy(acc.at[pl.ds(9600, 400)],
                                ox.at[pl.ds(9600, 400)])

        if p == 0:
            plsc.subcore_barrier()


_QSDS = jax.ShapeDtypeStruct((N, QC), jnp.float32)


@functools.partial(
    pl.kernel,
    out_type=(_QSDS,) * 4,
    mesh=_MESH,
    compiler_params=_UNTILED,
    scratch_types=[
        pltpu.VMEM((ANB, AEB), jnp.int32),
        pltpu.VMEM((ANB, AEB), jnp.int32),
    ] + [pltpu.VMEM((AEB, QC), jnp.float32)] * 8 + [
        pltpu.VMEM_SHARED((N, QC), jnp.float32),
    ] + [pltpu.SemaphoreType.DMA] * 16,
)
def _sc_agg(*args):
    _agg_body(*args)


def _dec_body(p_t, q_t, row3, col3, w2, b2v, out,
              ridx, cidx, w2v, b2s, pr0, qr0, pr1, qr1, outv,
              semp0, semq0, semp1, semq1):
    c = lax.axis_index("c")
    s = lax.axis_index("s")
    wid = s * NC + c

    pltpu.sync_copy(row3.at[wid], ridx)
    pltpu.sync_copy(col3.at[wid], cidx)
    pltpu.sync_copy(w2, w2v)
    pltpu.sync_copy(b2v, b2s)

    prs = (pr0, pr1)
    qrs = (qr0, qr1)
    psems = (semp0, semp1)
    qsems = (semq0, semq1)

    def start(j, b):
        pltpu.async_copy(p_t.at[ridx.at[pl.ds(j * EB, EB)]], prs[b], psems[b])
        pltpu.async_copy(q_t.at[cidx.at[pl.ds(j * EB, EB)]], qrs[b], qsems[b])

    def wait(b):
        pltpu.make_async_copy(p_t.at[ridx.at[pl.ds(0, EB)]],
                              prs[b], psems[b]).wait()
        pltpu.make_async_copy(q_t.at[cidx.at[pl.ds(0, EB)]],
                              qrs[b], qsems[b]).wait()

    lanes = lax.iota(jnp.int32, 16)

    def compute(j, pr, qr):
        def edge(e, vec):
            acc = b2s[...]
            for h in range(H // 16):
                pch = pr[e, pl.ds(h * 16, 16)]
                qch = qr[e, pl.ds(h * 16, 16)]
                g = jnp.maximum(pch + qch, 0.0)
                acc = acc + g * w2v[pl.ds(h * 16, 16)]
            lane = lax.rem(e, 16)
            vec = jnp.where(lanes == lane, jnp.sum(acc), vec)

            @pl.when(lane == 15)
            def _():
                outv[pl.ds(j * EB + e - 15, 16)] = vec

            return vec

        lax.fori_loop(0, EB, edge, jnp.zeros((16,), jnp.float32))

    start(0, 0)

    def pair(k, _):
        j0 = 2 * k
        start(j0 + 1, 1)
        wait(0)
        compute(j0, pr0, qr0)

        @pl.when(j0 + 2 < LBAT)
        def _():
            start(j0 + 2, 0)

        wait(1)
        compute(j0 + 1, pr1, qr1)
        return 0

    lax.fori_loop(0, LBAT // 2, pair, 0)
    pltpu.sync_copy(outv, out.at[pl.ds(wid * LPT, LPT)])


@functools.partial(
    pl.kernel,
    out_type=jax.ShapeDtypeStruct((LPAD,), jnp.float32),
    mesh=_MESH,
    compiler_params=pltpu.CompilerParams(needs_layout_passes=False),
    scratch_types=[
        pltpu.VMEM((LPT,), jnp.int32),
        pltpu.VMEM((LPT,), jnp.int32),
        pltpu.VMEM((H,), jnp.float32),
        pltpu.VMEM((16,), jnp.float32),
        pltpu.VMEM((EB, H), jnp.float32),
        pltpu.VMEM((EB, H), jnp.float32),
        pltpu.VMEM((EB, H), jnp.float32),
        pltpu.VMEM((EB, H), jnp.float32),
        pltpu.VMEM((LPT,), jnp.float32),
        pltpu.SemaphoreType.DMA,
        pltpu.SemaphoreType.DMA,
        pltpu.SemaphoreType.DMA,
        pltpu.SemaphoreType.DMA,
    ],
)
def _sc_decoder(p_t, q_t, row3, col3, w2, b2v, out,
                ridx, cidx, w2v, b2s, pr0, qr0, pr1, qr1, outv,
                semp0, semq0, semp1, semq1):
    _dec_body(p_t, q_t, row3, col3, w2, b2v, out,
              ridx, cidx, w2v, b2s, pr0, qr0, pr1, qr1, outv,
              semp0, semq0, semp1, semq1)


_RB = 1000


def _dot(a, b):
    return jnp.dot(a, b, preferred_element_type=jnp.float32)


_QBLK = pl.BlockSpec((_RB, QC), lambda i: (i, 0))
_WBLK = pl.BlockSpec((H, H), lambda i: (0, 0))
_BBLK = pl.BlockSpec((1, H), lambda i: (0, 0))
_QOUT = (_QBLK, _QBLK, _QBLK, _QBLK)
_QSHAPE = tuple(jax.ShapeDtypeStruct((N, QC), jnp.float32) for _ in range(4))


def _split4(out_refs, y):
    for k in range(4):
        out_refs[k][...] = y[:, k * QC:(k + 1) * QC]


def _proj_body(x_ref, w_ref, b_ref, *out_refs):
    _split4(out_refs, _dot(x_ref[...], w_ref[...]) + b_ref[...])


def _tc_proj(x, w, b):
    return pl.pallas_call(
        _proj_body,
        grid=(N // _RB,),
        in_specs=[
            pl.BlockSpec((_RB, H), lambda i: (i, 0)),
            _WBLK,
            _BBLK,
        ],
        out_specs=_QOUT,
        out_shape=_QSHAPE,
    )(x, w, b)


def _sage_z(agg_refs, cnt_ref, x_refs, wl_ref, bl_ref, wr_ref):
    inv = 1.0 / jnp.maximum(cnt_ref[...], 1.0)
    mean = jnp.concatenate([a[...] for a in agg_refs], axis=1) * inv
    xfull = jnp.concatenate([x[...] for x in x_refs], axis=1)
    return jnp.maximum(_dot(mean, wl_ref[...]) + bl_ref[...]
                       + _dot(xfull, wr_ref[...]), 0.0)


def _sage_body(a0, a1, a2, a3, cnt_ref, x0, x1, x2, x3,
               wl_ref, bl_ref, wr_ref, *out_refs):
    z = _sage_z((a0, a1, a2, a3), cnt_ref, (x0, x1, x2, x3),
                wl_ref, bl_ref, wr_ref)
    _split4(out_refs, z)


def _tc_sage(aggs, cnt, xs, wl, bl, wr):
    return pl.pallas_call(
        _sage_body,
        grid=(N // _RB,),
        in_specs=[_QBLK] * 4 + [pl.BlockSpec((_RB, 1), lambda i: (i, 0))]
                 + [_QBLK] * 4 + [_WBLK, _BBLK, _WBLK],
        out_specs=_QOUT,
        out_shape=_QSHAPE,
    )(*aggs, cnt, *xs, wl, bl, wr)


def _sage_dec_body(a0, a1, a2, a3, cnt_ref, x0, x1, x2, x3,
                   wl_ref, bl_ref, wr_ref, wd_ref, bd_ref, out_ref):
    z = _sage_z((a0, a1, a2, a3), cnt_ref, (x0, x1, x2, x3),
                wl_ref, bl_ref, wr_ref)
    out_ref[...] = _dot(z, wd_ref[...]) + bd_ref[...]


def _tc_sage_dec(aggs, cnt, xs, wl, bl, wr, wd, bd):
    return pl.pallas_call(
        _sage_dec_body,
        grid=(N // _RB,),
        in_specs=[_QBLK] * 4 + [pl.BlockSpec((_RB, 1), lambda i: (i, 0))]
                 + [_QBLK] * 4 + [_WBLK, _BBLK, _WBLK, _WBLK, _BBLK],
        out_specs=pl.BlockSpec((_RB, H), lambda i: (i, 0)),
        out_shape=jax.ShapeDtypeStruct((N, H), jnp.float32),
    )(*aggs, cnt, *xs, wl, bl, wr, wd, bd)


def kernel(x_drug, x_reaction, ei_drug_to_reaction, ei_reaction_rev_drug,
           edge_label_index,
           W_drug_lin, b_drug_lin, W_reaction_lin, b_reaction_lin,
           Wl1_dr, bl1_dr, Wr1_dr, Wl1_rd, bl1_rd, Wr1_rd,
           Wl2_dr, bl2_dr, Wr2_dr, Wl2_rd, bl2_rd, Wr2_rd,
           W_dec1, b_dec1, W_dec2, b_dec2):
    f32 = jnp.float32
    i32 = jnp.int32

    src_dr = ei_drug_to_reaction[0].astype(i32).reshape(NS, ANB, AEB)
    dst_dr = ei_drug_to_reaction[1].astype(i32).reshape(NS, ANB, AEB)
    src_rd = ei_reaction_rev_drug[0].astype(i32).reshape(NS, ANB, AEB)
    dst_rd = ei_reaction_rev_drug[1].astype(i32).reshape(NS, ANB, AEB)

    pad = jnp.zeros((LPAD - L,), i32)
    row3 = jnp.concatenate([edge_label_index[0].astype(i32), pad]
                           ).reshape(NC * NS, LPT)
    col3 = jnp.concatenate([edge_label_index[1].astype(i32), pad]
                           ).reshape(NC * NS, LPT)

    zeros_s = jnp.zeros((640, 16), f32)
    zeros_b = jnp.zeros((640, QC), f32)

    b_drug = b_drug_lin.reshape(1, H)
    b_react = b_reaction_lin.reshape(1, H)

    cnts = _sc_counts(dst_dr, dst_rd, zeros_s)
    cnt_r = cnts[0, :, 0:1]
    cnt_d = cnts[1, :, 0:1]

    xd = _tc_proj(x_drug, W_drug_lin, b_drug)
    xr = _tc_proj(x_reaction, W_reaction_lin, b_react)

    agg_r1 = _sc_agg(*xd, src_dr, dst_dr, zeros_b)
    agg_d1 = _sc_agg(*xr, src_rd, dst_rd, zeros_b)
    zr = _tc_sage(agg_r1, cnt_r, xr, Wl1_dr, bl1_dr.reshape(1, H), Wr1_dr)
    zd = _tc_sage(agg_d1, cnt_d, xd, Wl1_rd, bl1_rd.reshape(1, H), Wr1_rd)

    agg_r2 = _sc_agg(*zd, src_dr, dst_dr, zeros_b)
    agg_d2 = _sc_agg(*zr, src_rd, dst_rd, zeros_b)
    p_t = _tc_sage_dec(agg_d2, cnt_d, zd, Wl2_rd, bl2_rd.reshape(1, H),
                       Wr2_rd, W_dec1[:H], b_dec1.reshape(1, H))
    q_t = _tc_sage_dec(agg_r2, cnt_r, zr, Wl2_dr, bl2_dr.reshape(1, H),
                       Wr2_dr, W_dec1[H:], jnp.zeros((1, H), f32))

    w2 = W_dec2[:, 0]
    b2v = jnp.zeros((16,), f32).at[0].set(b_dec2[0])
    out = _sc_decoder(p_t, q_t, row3, col3, w2, b2v)
    return out[:L]


# speedup vs baseline: 1.3118x; 1.0128x over previous
import functools

import jax
import jax.numpy as jnp
from jax import lax
from jax.experimental import pallas as pl
from jax.experimental.pallas import tpu as pltpu
import jax.experimental.pallas.tpu_sc as plsc

N = 10000
H = 256
HC = 128
E = 160000
L = 40000
NC, NS = 2, 16
EB = 80
EPT = E // NS
AEB = 125
ANB = EPT // AEB
NBAT = EPT // EB
RCH = 80
NRCH = N // RCH
LPAD = 40960
LPT = LPAD // (NC * NS)
LBAT = LPT // EB

_MESH = plsc.VectorSubcoreMesh(core_axis_name="c", subcore_axis_name="s")
_UNTILED = pltpu.CompilerParams(use_tc_tiling_on_sc=False)


def _count_body(dst_a, dst_b, zeros_s, out, didx, ones_v, hist):
    c = lax.axis_index("c")
    s = lax.axis_index("s")

    def fill_ones(i, _):
        ones_v[i, :] = jnp.ones((16,), jnp.float32)
        return 0

    lax.fori_loop(0, AEB, fill_ones, 0)

    @pl.when(c == 0)
    def _():
        pltpu.sync_copy(dst_a.at[s], didx)

    @pl.when(c == 1)
    def _():
        pltpu.sync_copy(dst_b.at[s], didx)

    @pl.when(s < NS - 1)
    def _():
        pltpu.sync_copy(zeros_s, hist.at[pl.ds(s * 640, 640)])

    @pl.when(s == NS - 1)
    def _():
        pltpu.sync_copy(zeros_s.at[pl.ds(0, 400)], hist.at[pl.ds(9600, 400)])

    plsc.subcore_barrier()

    def scat(i, _):
        pltpu.sync_copy(ones_v, hist.at[didx.at[i]], add=True)
        return 0

    lax.fori_loop(0, ANB, scat, 0)
    plsc.subcore_barrier()

    @pl.when(s < NS - 1)
    def _():
        pltpu.sync_copy(hist.at[pl.ds(s * 640, 640)],
                        out.at[c, pl.ds(s * 640, 640)])

    @pl.when(s == NS - 1)
    def _():
        pltpu.sync_copy(hist.at[pl.ds(9600, 400)],
                        out.at[c, pl.ds(9600, 400)])


@functools.partial(
    pl.kernel,
    out_type=jax.ShapeDtypeStruct((NC, N, 16), jnp.float32),
    mesh=_MESH,
    compiler_params=_UNTILED,
    scratch_types=[
        pltpu.VMEM((ANB, AEB), jnp.int32),
        pltpu.VMEM((AEB, 16), jnp.float32),
        pltpu.VMEM_SHARED((N, 16), jnp.float32),
    ],
)
def _sc_counts(dst_a, dst_b, zeros_s, out, didx, ones_v, hist):
    _count_body(dst_a, dst_b, zeros_s, out, didx, ones_v, hist)


QC = 64


def _agg_body(x0, x1, x2, x3, src3, dst3, zeros_b, out0, out1, out2, out3,
              sidx, didx, *rest):
    bufs = rest[0:8]
    acc = rest[8]
    gsems = rest[9:17]
    ssems = rest[17:25]
    nb = len(bufs)
    c = lax.axis_index("c")
    s = lax.axis_index("s")

    pltpu.sync_copy(src3.at[s], sidx)
    pltpu.sync_copy(dst3.at[s], didx)

    def zero_acc():
        @pl.when(s < NS - 1)
        def _():
            pltpu.sync_copy(zeros_b, acc.at[pl.ds(s * 640, 640)])

        @pl.when(s == NS - 1)
        def _():
            pltpu.sync_copy(zeros_b.at[pl.ds(0, 400)],
                            acc.at[pl.ds(9600, 400)])

    phases = ((x0, x2, out0, out2), (x1, x3, out1, out3))

    def mk_gather(xa, xb):
        def start_gather(i, buf, gsem):
            @pl.when(c == 0)
            def _():
                pltpu.async_copy(xa.at[sidx.at[i]], buf, gsem)

            @pl.when(c == 1)
            def _():
                pltpu.async_copy(xb.at[sidx.at[i]], buf, gsem)

        def wait_gather(buf, gsem):
            pltpu.make_async_copy(xa.at[sidx.at[0]], buf, gsem).wait()

        return start_gather, wait_gather

    def start_scatter(i, buf, ssem):
        pltpu.async_copy(buf, acc.at[didx.at[i]], ssem, add=True)

    def wait_scatter(i, buf, ssem):
        pltpu.make_async_copy(buf, acc.at[didx.at[i]], ssem).wait()

    zero_acc()
    plsc.subcore_barrier()
    for p, (xa, xb, oa, ob) in enumerate(phases):
        start_gather, wait_gather = mk_gather(xa, xb)
        if p == 0:
            for b in range(nb):
                start_gather(b, bufs[b], gsems[b])

        def bodyn(k, _):
            i0 = nb * k
            for b in range(nb):
                wait_gather(bufs[b], gsems[b])
                start_scatter(i0 + b, bufs[b], ssems[b])
            for b in range(nb):
                wait_scatter(i0 + b, bufs[b], ssems[b])

                @pl.when(i0 + b + nb < ANB)
                def _():
                    start_gather(i0 + b + nb, bufs[b], gsems[b])

            return 0

        lax.fori_loop(0, ANB // nb, bodyn, 0)
        plsc.subcore_barrier()

        if p == 0:
            sg1, _wg1 = mk_gather(phases[1][0], phases[1][1])
            for b in range(nb):
                sg1(b, bufs[b], gsems[b])

        for cc, ox in ((0, oa), (1, ob)):
            @pl.when((c == cc) & (s < NS - 1))
            def _():
                pltpu.sync_copy(acc.at[pl.ds(s * 640, 640)],
                                ox.at[pl.ds(s * 640, 640)])

            @pl.when((c == cc) & (s == NS - 1))
            def _():
                pltpu.sync_copy(acc.at[pl.ds(9600, 400)],
                                ox.at[pl.ds(9600, 400)])

        if p == 0:
            zero_acc()
            plsc.subcore_barrier()


_QSDS = jax.ShapeDtypeStruct((N, QC), jnp.float32)


@functools.partial(
    pl.kernel,
    out_type=(_QSDS,) * 4,
    mesh=_MESH,
    compiler_params=_UNTILED,
    scratch_types=[
        pltpu.VMEM((ANB, AEB), jnp.int32),
        pltpu.VMEM((ANB, AEB), jnp.int32),
    ] + [pltpu.VMEM((AEB, QC), jnp.float32)] * 8 + [
        pltpu.VMEM_SHARED((N, QC), jnp.float32),
    ] + [pltpu.SemaphoreType.DMA] * 16,
)
def _sc_agg(*args):
    _agg_body(*args)


def _dec_body(p_t, q_t, row3, col3, w2, b2v, out,
              ridx, cidx, w2v, b2s, pr0, qr0, pr1, qr1, outv,
              semp0, semq0, semp1, semq1):
    c = lax.axis_index("c")
    s = lax.axis_index("s")
    wid = s * NC + c

    pltpu.sync_copy(row3.at[wid], ridx)
    pltpu.sync_copy(col3.at[wid], cidx)
    pltpu.sync_copy(w2, w2v)
    pltpu.sync_copy(b2v, b2s)

    prs = (pr0, pr1)
    qrs = (qr0, qr1)
    psems = (semp0, semp1)
    qsems = (semq0, semq1)

    def start(j, b):
        pltpu.async_copy(p_t.at[ridx.at[pl.ds(j * EB, EB)]], prs[b], psems[b])
        pltpu.async_copy(q_t.at[cidx.at[pl.ds(j * EB, EB)]], qrs[b], qsems[b])

    def wait(b):
        pltpu.make_async_copy(p_t.at[ridx.at[pl.ds(0, EB)]],
                              prs[b], psems[b]).wait()
        pltpu.make_async_copy(q_t.at[cidx.at[pl.ds(0, EB)]],
                              qrs[b], qsems[b]).wait()

    lanes = lax.iota(jnp.int32, 16)

    def compute(j, pr, qr):
        def edge(e, vec):
            acc = b2s[...]
            for h in range(H // 16):
                pch = pr[e, pl.ds(h * 16, 16)]
                qch = qr[e, pl.ds(h * 16, 16)]
                g = jnp.maximum(pch + qch, 0.0)
                acc = acc + g * w2v[pl.ds(h * 16, 16)]
            lane = lax.rem(e, 16)
            vec = jnp.where(lanes == lane, jnp.sum(acc), vec)

            @pl.when(lane == 15)
            def _():
                outv[pl.ds(j * EB + e - 15, 16)] = vec

            return vec

        lax.fori_loop(0, EB, edge, jnp.zeros((16,), jnp.float32))

    start(0, 0)

    def pair(k, _):
        j0 = 2 * k
        start(j0 + 1, 1)
        wait(0)
        compute(j0, pr0, qr0)

        @pl.when(j0 + 2 < LBAT)
        def _():
            start(j0 + 2, 0)

        wait(1)
        compute(j0 + 1, pr1, qr1)
        return 0

    lax.fori_loop(0, LBAT // 2, pair, 0)
    pltpu.sync_copy(outv, out.at[pl.ds(wid * LPT, LPT)])


@functools.partial(
    pl.kernel,
    out_type=jax.ShapeDtypeStruct((LPAD,), jnp.float32),
    mesh=_MESH,
    compiler_params=pltpu.CompilerParams(needs_layout_passes=False),
    scratch_types=[
        pltpu.VMEM((LPT,), jnp.int32),
        pltpu.VMEM((LPT,), jnp.int32),
        pltpu.VMEM((H,), jnp.float32),
        pltpu.VMEM((16,), jnp.float32),
        pltpu.VMEM((EB, H), jnp.float32),
        pltpu.VMEM((EB, H), jnp.float32),
        pltpu.VMEM((EB, H), jnp.float32),
        pltpu.VMEM((EB, H), jnp.float32),
        pltpu.VMEM((LPT,), jnp.float32),
        pltpu.SemaphoreType.DMA,
        pltpu.SemaphoreType.DMA,
        pltpu.SemaphoreType.DMA,
        pltpu.SemaphoreType.DMA,
    ],
)
def _sc_decoder(p_t, q_t, row3, col3, w2, b2v, out,
                ridx, cidx, w2v, b2s, pr0, qr0, pr1, qr1, outv,
                semp0, semq0, semp1, semq1):
    _dec_body(p_t, q_t, row3, col3, w2, b2v, out,
              ridx, cidx, w2v, b2s, pr0, qr0, pr1, qr1, outv,
              semp0, semq0, semp1, semq1)


_RB = 1000


def _dot(a, b):
    return jnp.dot(a, b, preferred_element_type=jnp.float32)


_QBLK = pl.BlockSpec((_RB, QC), lambda i: (i, 0))
_WBLK = pl.BlockSpec((H, H), lambda i: (0, 0))
_BBLK = pl.BlockSpec((1, H), lambda i: (0, 0))
_QOUT = (_QBLK, _QBLK, _QBLK, _QBLK)
_QSHAPE = tuple(jax.ShapeDtypeStruct((N, QC), jnp.float32) for _ in range(4))


def _split4(out_refs, y):
    for k in range(4):
        out_refs[k][...] = y[:, k * QC:(k + 1) * QC]


def _proj_body(x_ref, w_ref, b_ref, *out_refs):
    _split4(out_refs, _dot(x_ref[...], w_ref[...]) + b_ref[...])


def _tc_proj(x, w, b):
    return pl.pallas_call(
        _proj_body,
        grid=(N // _RB,),
        in_specs=[
            pl.BlockSpec((_RB, H), lambda i: (i, 0)),
            _WBLK,
            _BBLK,
        ],
        out_specs=_QOUT,
        out_shape=_QSHAPE,
    )(x, w, b)


def _sage_z(agg_refs, cnt_ref, x_refs, wl_ref, bl_ref, wr_ref):
    inv = 1.0 / jnp.maximum(cnt_ref[...], 1.0)
    mean = jnp.concatenate([a[...] for a in agg_refs], axis=1) * inv
    xfull = jnp.concatenate([x[...] for x in x_refs], axis=1)
    return jnp.maximum(_dot(mean, wl_ref[...]) + bl_ref[...]
                       + _dot(xfull, wr_ref[...]), 0.0)


def _sage_body(a0, a1, a2, a3, cnt_ref, x0, x1, x2, x3,
               wl_ref, bl_ref, wr_ref, *out_refs):
    z = _sage_z((a0, a1, a2, a3), cnt_ref, (x0, x1, x2, x3),
                wl_ref, bl_ref, wr_ref)
    _split4(out_refs, z)


def _tc_sage(aggs, cnt, xs, wl, bl, wr):
    return pl.pallas_call(
        _sage_body,
        grid=(N // _RB,),
        in_specs=[_QBLK] * 4 + [pl.BlockSpec((_RB, 1), lambda i: (i, 0))]
                 + [_QBLK] * 4 + [_WBLK, _BBLK, _WBLK],
        out_specs=_QOUT,
        out_shape=_QSHAPE,
    )(*aggs, cnt, *xs, wl, bl, wr)


def _sage_dec_body(a0, a1, a2, a3, cnt_ref, x0, x1, x2, x3,
                   wl_ref, bl_ref, wr_ref, wd_ref, bd_ref, out_ref):
    z = _sage_z((a0, a1, a2, a3), cnt_ref, (x0, x1, x2, x3),
                wl_ref, bl_ref, wr_ref)
    out_ref[...] = _dot(z, wd_ref[...]) + bd_ref[...]


def _tc_sage_dec(aggs, cnt, xs, wl, bl, wr, wd, bd):
    return pl.pallas_call(
        _sage_dec_body,
        grid=(N // _RB,),
        in_specs=[_QBLK] * 4 + [pl.BlockSpec((_RB, 1), lambda i: (i, 0))]
                 + [_QBLK] * 4 + [_WBLK, _BBLK, _WBLK, _WBLK, _BBLK],
        out_specs=pl.BlockSpec((_RB, H), lambda i: (i, 0)),
        out_shape=jax.ShapeDtypeStruct((N, H), jnp.float32),
    )(*aggs, cnt, *xs, wl, bl, wr, wd, bd)


def kernel(x_drug, x_reaction, ei_drug_to_reaction, ei_reaction_rev_drug,
           edge_label_index,
           W_drug_lin, b_drug_lin, W_reaction_lin, b_reaction_lin,
           Wl1_dr, bl1_dr, Wr1_dr, Wl1_rd, bl1_rd, Wr1_rd,
           Wl2_dr, bl2_dr, Wr2_dr, Wl2_rd, bl2_rd, Wr2_rd,
           W_dec1, b_dec1, W_dec2, b_dec2):
    f32 = jnp.float32
    i32 = jnp.int32

    src_dr = ei_drug_to_reaction[0].astype(i32).reshape(NS, ANB, AEB)
    dst_dr = ei_drug_to_reaction[1].astype(i32).reshape(NS, ANB, AEB)
    src_rd = ei_reaction_rev_drug[0].astype(i32).reshape(NS, ANB, AEB)
    dst_rd = ei_reaction_rev_drug[1].astype(i32).reshape(NS, ANB, AEB)

    pad = jnp.zeros((LPAD - L,), i32)
    row3 = jnp.concatenate([edge_label_index[0].astype(i32), pad]
                           ).reshape(NC * NS, LPT)
    col3 = jnp.concatenate([edge_label_index[1].astype(i32), pad]
                           ).reshape(NC * NS, LPT)

    zeros_s = jnp.zeros((640, 16), f32)
    zeros_b = jnp.zeros((640, QC), f32)

    b_drug = b_drug_lin.reshape(1, H)
    b_react = b_reaction_lin.reshape(1, H)

    cnts = _sc_counts(dst_dr, dst_rd, zeros_s)
    cnt_r = cnts[0, :, 0:1]
    cnt_d = cnts[1, :, 0:1]

    xd = _tc_proj(x_drug, W_drug_lin, b_drug)
    xr = _tc_proj(x_reaction, W_reaction_lin, b_react)

    agg_r1 = _sc_agg(*xd, src_dr, dst_dr, zeros_b)
    agg_d1 = _sc_agg(*xr, src_rd, dst_rd, zeros_b)
    zr = _tc_sage(agg_r1, cnt_r, xr, Wl1_dr, bl1_dr.reshape(1, H), Wr1_dr)
    zd = _tc_sage(agg_d1, cnt_d, xd, Wl1_rd, bl1_rd.reshape(1, H), Wr1_rd)

    agg_r2 = _sc_agg(*zd, src_dr, dst_dr, zeros_b)
    agg_d2 = _sc_agg(*zr, src_rd, dst_rd, zeros_b)
    p_t = _tc_sage_dec(agg_d2, cnt_d, zd, Wl2_rd, bl2_rd.reshape(1, H),
                       Wr2_rd, W_dec1[:H], b_dec1.reshape(1, H))
    q_t = _tc_sage_dec(agg_r2, cnt_r, zr, Wl2_dr, bl2_dr.reshape(1, H),
                       Wr2_dr, W_dec1[H:], jnp.zeros((1, H), f32))

    w2 = W_dec2[:, 0]
    b2v = jnp.zeros((16,), f32).at[0].set(b_dec2[0])
    out = _sc_decoder(p_t, q_t, row3, col3, w2, b2v)
    return out[:L]
